# Initial kernel scaffold; baseline (speedup 1.0000x reference)
import jax
import jax.numpy as jnp
from jax.experimental import pallas as pl


def kernel(x, W_enc, b_enc, W_dec, b_dec):
    raise NotImplementedError("write your pallas kernel here")


# speedup vs baseline: 4.2287x; 4.2287x over previous
import functools

import jax
import jax.numpy as jnp
from jax.experimental import pallas as pl
from jax.experimental.pallas import tpu as pltpu

_ROWS = 128
_IN = 2048
_SAE = 32768
_K = 64
_BLK = 2048


def _encode_kernel(x_ref, w_ref, b_ref, out_ref):
    acc = jax.lax.dot_general(
        x_ref[...], w_ref[...],
        dimension_numbers=(((1,), (1,)), ((), ())),
        preferred_element_type=jnp.float32,
    )
    out_ref[...] = jnp.maximum(acc + b_ref[...], 0.0)


def _thresh_kernel(pre_ref, t_ref):
    bits = jax.lax.bitcast_convert_type(pre_ref[...], jnp.int32)

    def body(i, t):
        cand = t | (jnp.int32(1) << (30 - i))
        cnt = jnp.sum((bits >= cand).astype(jnp.int32), axis=1, keepdims=True)
        return jnp.where(cnt >= _K, cand, t)

    t = jax.lax.fori_loop(0, 31, body, jnp.zeros((_ROWS, 1), jnp.int32))
    t_ref[...] = jax.lax.bitcast_convert_type(t, jnp.float32)


def _decode_kernel(pre_ref, t_ref, wd_ref, bd_ref, sparse_ref, recon_ref,
                   acc_ref, *, nblk):
    i = pl.program_id(0)
    pa = pre_ref[...]
    s = jnp.where(pa >= t_ref[...], pa, 0.0)
    sparse_ref[...] = s
    part = jax.lax.dot_general(
        s, wd_ref[...],
        dimension_numbers=(((1,), (1,)), ((), ())),
        preferred_element_type=jnp.float32,
    )

    @pl.when(i == 0)
    def _():
        acc_ref[...] = part

    @pl.when(i > 0)
    def _():
        acc_ref[...] += part

    @pl.when(i == nblk - 1)
    def _():
        recon_ref[...] = acc_ref[...] + bd_ref[...]


def kernel(x, W_enc, b_enc, W_dec, b_dec):
    nblk = _SAE // _BLK
    b_enc2 = b_enc.reshape(1, _SAE)
    b_dec2 = b_dec.reshape(1, _IN)

    pre_act = pl.pallas_call(
        _encode_kernel,
        grid=(nblk,),
        in_specs=[
            pl.BlockSpec((_ROWS, _IN), lambda i: (0, 0)),
            pl.BlockSpec((_BLK, _IN), lambda i: (i, 0)),
            pl.BlockSpec((1, _BLK), lambda i: (0, i)),
        ],
        out_specs=pl.BlockSpec((_ROWS, _BLK), lambda i: (0, i)),
        out_shape=jax.ShapeDtypeStruct((_ROWS, _SAE), jnp.float32),
    )(x, W_enc, b_enc2)

    thresh = pl.pallas_call(
        _thresh_kernel,
        out_shape=jax.ShapeDtypeStruct((_ROWS, 1), jnp.float32),
    )(pre_act)

    sparse_act, recon = pl.pallas_call(
        functools.partial(_decode_kernel, nblk=nblk),
        grid=(nblk,),
        in_specs=[
            pl.BlockSpec((_ROWS, _BLK), lambda i: (0, i)),
            pl.BlockSpec((_ROWS, 1), lambda i: (0, 0)),
            pl.BlockSpec((_IN, _BLK), lambda i: (0, i)),
            pl.BlockSpec((1, _IN), lambda i: (0, 0)),
        ],
        out_specs=[
            pl.BlockSpec((_ROWS, _BLK), lambda i: (0, i)),
            pl.BlockSpec((_ROWS, _IN), lambda i: (0, 0)),
        ],
        out_shape=[
            jax.ShapeDtypeStruct((_ROWS, _SAE), jnp.float32),
            jax.ShapeDtypeStruct((_ROWS, _IN), jnp.float32),
        ],
        scratch_shapes=[pltpu.VMEM((_ROWS, _IN), jnp.float32)],
    )(pre_act, thresh, W_dec, b_dec2)

    return (recon, sparse_act)
